# baseline (speedup 1.0000x reference)
import jax
import jax.numpy as jnp
from jax import lax
from jax.experimental import pallas as pl
from jax.experimental.pallas import tpu as pltpu
from jax.experimental.pallas import tpu_sc as plsc

N_HID = 256
E = 160000
NC = 2
NS = 16
NW = NC * NS
BPW = E // NW
CH = 128
NCH = -(-BPW // CH)
TS = BPW - (NCH - 1) * CH
NBUF = 2
PF = 1


def _fuse_body(emb_ref, w_ref, b_ref, out_ref):
    out_ref[...] = lax.dot_general(
        emb_ref[...], w_ref[...],
        (((1,), (1,)), ((), ())),
        preferred_element_type=jnp.float32,
        precision=lax.Precision.HIGHEST,
    ) + b_ref[...]


def _fuse_table(emb_table, W, b):
    m, n = emb_table.shape
    return pl.pallas_call(
        _fuse_body,
        out_shape=jax.ShapeDtypeStruct((m, n), jnp.float32),
    )(emb_table, W, b.reshape(1, n))


def _gather_body(table_hbm, idx_hbm, out_hbm, idx_v, rows_v, gs, ws):
    wid = lax.axis_index("s") * NC + lax.axis_index("c")
    base = pl.multiple_of(wid * BPW, 8)
    pltpu.sync_copy(idx_hbm.at[wid], idx_v)

    def gather(c):
        b = c % NBUF
        return pltpu.make_async_copy(
            table_hbm.at[idx_v.at[c]], rows_v.at[b], gs[b])

    def write(c):
        b = c % NBUF
        n = TS if c == NCH - 1 else CH
        return pltpu.make_async_copy(
            rows_v.at[b, pl.ds(0, n)],
            out_hbm.at[pl.ds(pl.multiple_of(base + c * CH, 8), n)], ws[b])

    for c in range(min(PF, NCH)):
        gather(c).start()
    for c in range(NCH):
        pc = c + PF
        if pc < NCH:
            if pc - NBUF >= 0:
                write(pc - NBUF).wait()
            gather(pc).start()
        gather(c).wait()
        write(c).start()
    for c in range(max(0, NCH - NBUF), NCH):
        write(c).wait()


def _sc_gather(table, idx):
    mesh = plsc.VectorSubcoreMesh(
        core_axis_name="c", subcore_axis_name="s",
        num_cores=NC, num_subcores=NS)
    return pl.kernel(
        _gather_body,
        out_type=jax.ShapeDtypeStruct((E, N_HID), jnp.float32),
        mesh=mesh,
        scratch_types=[
            pltpu.VMEM((NCH, CH), jnp.int32),
            pltpu.VMEM((NBUF, CH, N_HID), jnp.float32),
            [pltpu.SemaphoreType.DMA] * NBUF,
            [pltpu.SemaphoreType.DMA] * NBUF,
        ],
    )(table, idx)


def kernel(t, emb_table, W, b):
    fused = _fuse_table(emb_table, W, b)
    idx = jnp.pad(t.reshape(NW, BPW), ((0, 0), (0, NCH * CH - BPW)))
    idx = idx.reshape(NW, NCH, CH)
    return _sc_gather(fused, idx)


# speedup vs baseline: 1.3252x; 1.3252x over previous
import jax
import jax.numpy as jnp
from jax import lax
from jax.experimental import pallas as pl
from jax.experimental.pallas import tpu as pltpu
from jax.experimental.pallas import tpu_sc as plsc

N_HID = 256
E = 160000
NC = 2
NS = 16
NW = NC * NS
BPW = E // NW
CH = 112
NCH = -(-BPW // CH)
TS = BPW - (NCH - 1) * CH
NBUF = 4
PF = 2


def _fuse_body(emb_ref, w_ref, b_ref, out_ref):
    out_ref[...] = lax.dot_general(
        emb_ref[...], w_ref[...],
        (((1,), (1,)), ((), ())),
        preferred_element_type=jnp.float32,
        precision=lax.Precision.HIGHEST,
    ) + b_ref[...]


def _fuse_table(emb_table, W, b):
    m, n = emb_table.shape
    return pl.pallas_call(
        _fuse_body,
        out_shape=jax.ShapeDtypeStruct((m, n), jnp.float32),
    )(emb_table, W, b.reshape(1, n))


def _gather_body(table_hbm, idx_hbm, out_hbm, idx_v, rows_v, gs, ws):
    wid = lax.axis_index("s") * NC + lax.axis_index("c")
    base = pl.multiple_of(wid * BPW, 8)
    pltpu.sync_copy(idx_hbm.at[wid], idx_v)

    def gather(c):
        b = c % NBUF
        return pltpu.make_async_copy(
            table_hbm.at[idx_v.at[c]], rows_v.at[b], gs[b])

    def write(c):
        b = c % NBUF
        n = TS if c == NCH - 1 else CH
        return pltpu.make_async_copy(
            rows_v.at[b, pl.ds(0, n)],
            out_hbm.at[pl.ds(pl.multiple_of(base + c * CH, 8), n)], ws[b])

    for c in range(min(PF, NCH)):
        gather(c).start()
    for c in range(NCH):
        pc = c + PF
        if pc < NCH:
            if pc - NBUF >= 0:
                write(pc - NBUF).wait()
            gather(pc).start()
        gather(c).wait()
        write(c).start()
    for c in range(max(0, NCH - NBUF), NCH):
        write(c).wait()


def _sc_gather(table, idx):
    mesh = plsc.VectorSubcoreMesh(
        core_axis_name="c", subcore_axis_name="s",
        num_cores=NC, num_subcores=NS)
    return pl.kernel(
        _gather_body,
        out_type=jax.ShapeDtypeStruct((E, N_HID), jnp.float32),
        mesh=mesh,
        scratch_types=[
            pltpu.VMEM((NCH, CH), jnp.int32),
            pltpu.VMEM((NBUF, CH, N_HID), jnp.float32),
            [pltpu.SemaphoreType.DMA] * NBUF,
            [pltpu.SemaphoreType.DMA] * NBUF,
        ],
    )(table, idx)


def kernel(t, emb_table, W, b):
    fused = _fuse_table(emb_table, W, b)
    idx = jnp.pad(t.reshape(NW, BPW), ((0, 0), (0, NCH * CH - BPW)))
    idx = idx.reshape(NW, NCH, CH)
    return _sc_gather(fused, idx)
